# baseline (speedup 1.0000x reference)
import functools

import jax
import jax.numpy as jnp
from jax import lax
from jax.experimental import pallas as pl
from jax.experimental.pallas import tpu as pltpu
from jax.experimental.pallas import tpu_sc as plsc

_B = 16384
_D = 64
_NW = 32
_BPW = _B // _NW
_CH = 128
_NCH = _BPW // _CH
_W = 8
_L = 16

_mesh = plsc.VectorSubcoreMesh(core_axis_name="c", subcore_axis_name="s")


@functools.partial(
    pl.kernel,
    mesh=_mesh,
    compiler_params=pltpu.CompilerParams(
        use_tc_tiling_on_sc=False, needs_layout_passes=False),
    out_type=[
        jax.ShapeDtypeStruct((_NW * _NCH, _CH, _D), jnp.float32),
        jax.ShapeDtypeStruct((_NW * _NCH, _CH), jnp.float32),
        jax.ShapeDtypeStruct((_NW * _NCH, _CH), jnp.float32),
    ],
    scratch_types=[
        pltpu.VMEM((_NCH, _CH), jnp.int32),
        pltpu.VMEM((_NCH, _CH), jnp.int32),
        pltpu.VMEM((_NCH, _CH, _D), jnp.float32),
        pltpu.VMEM((_NCH, _CH, _W), jnp.float32),
        pltpu.VMEM((_NCH, _CH, _W), jnp.float32),
        pltpu.VMEM((_NCH, _CH), jnp.float32),
        pltpu.VMEM((_NCH, _CH), jnp.float32),
        pltpu.SemaphoreType.DMA,
        pltpu.SemaphoreType.DMA,
    ],
)
def _gather_all(idx_hbm, hi_hbm, emb_hbm, inh_hbm, det_hbm,
                z_out, inh_out, det_out,
                idx_v, hi_v, rows_v, inh_rows, det_rows,
                inh_sel, det_sel, sem, sem2):
    wid = lax.axis_index("s") * 2 + lax.axis_index("c")
    base = wid * _NCH
    pltpu.sync_copy(idx_hbm.at[pl.ds(base, _NCH)], idx_v)
    pltpu.sync_copy(hi_hbm.at[pl.ds(base, _NCH)], hi_v)
    copies = []
    for j in range(_NCH):
        copies.append(pltpu.async_copy(emb_hbm.at[idx_v.at[j]], rows_v.at[j], sem))
        copies.append(pltpu.async_copy(inh_hbm.at[hi_v.at[j]], inh_rows.at[j], sem))
        copies.append(pltpu.async_copy(det_hbm.at[hi_v.at[j]], det_rows.at[j], sem))
    for c in copies:
        c.wait()
    zcopy = pltpu.async_copy(rows_v, z_out.at[pl.ds(base, _NCH)], sem2)
    iotas = [lax.iota(jnp.int32, _L) + (_L * t) for t in range(_CH // _L)]
    for j in range(_NCH):
        for t in range(_CH // _L):
            o = _L * t
            v = idx_v[j, pl.ds(o, _L)]
            lo = lax.bitwise_and(v, 7)
            row = iotas[t]
            inh_sel[j, pl.ds(o, _L)] = plsc.load_gather(
                inh_rows.at[j], [row, lo])
            det_sel[j, pl.ds(o, _L)] = plsc.load_gather(
                det_rows.at[j], [row, lo])
    pltpu.sync_copy(inh_sel, inh_out.at[pl.ds(base, _NCH)])
    pltpu.sync_copy(det_sel, det_out.at[pl.ds(base, _NCH)])
    zcopy.wait()


def kernel(piano_model, embedding, inharm_embedding, detune_embedding):
    idx = piano_model.astype(jnp.int32)
    idx2d = idx.reshape(_NW * _NCH, _CH)
    hi2d = (idx >> 3).reshape(_NW * _NCH, _CH)
    z, inh, det = _gather_all(
        idx2d, hi2d, embedding,
        inharm_embedding.reshape(-1, _W),
        detune_embedding.reshape(-1, _W))
    return (z.reshape(_B, 1, _D),
            inh.reshape(_B, 1, 1),
            det.reshape(_B, 1, 1))
